# baseline (speedup 1.0000x reference)
import functools

import jax
import jax.numpy as jnp
from jax import lax
from jax.experimental import pallas as pl
from jax.experimental.pallas import tpu as pltpu
import jax.experimental.pallas.tpu_sc as plsc

_NC = 2
_NS = 16
_LANES = 16
_NSLOT = 3
_H = 2


def kernel(x, token_embedding, pos_embedding):
    B, S = x.shape
    V, D = token_embedding.shape
    NW = _NC * _NS
    rows_per_w = B // NW
    ids_per_w = rows_per_w * S
    splits = ((0, 128), (128, S - 128))

    x_flat = x.astype(jnp.int32).reshape(-1)
    pos_t = pos_embedding.reshape(S // 8, 8, D)

    mesh = plsc.VectorSubcoreMesh(core_axis_name="c", subcore_axis_name="s")

    @functools.partial(
        pl.kernel,
        out_type=jax.ShapeDtypeStruct((B, S, D), jnp.float32),
        mesh=mesh,
        scratch_types=[
            pltpu.VMEM((ids_per_w,), jnp.int32),
            pltpu.VMEM((_NSLOT, S, D), jnp.float32),
            pltpu.VMEM((S // 8, 8, D), jnp.float32),
            [pltpu.SemaphoreType.DMA] * _NSLOT,
            [pltpu.SemaphoreType.DMA] * _NSLOT,
        ],
    )
    def emb(x_hbm, tok_hbm, pos_hbm, out_hbm, idx_all, rows_v, pos_v, gsem, osem):
        wid = lax.axis_index("s") * _NC + lax.axis_index("c")
        base_row = wid * rows_per_w
        pltpu.sync_copy(pos_hbm, pos_v)
        pltpu.sync_copy(x_hbm.at[pl.ds(wid * ids_per_w, ids_per_w)], idx_all)

        def gather_descs(r, slot):
            return [
                pltpu.make_async_copy(
                    tok_hbm.at[idx_all.at[pl.ds(r * S + off, ln)]],
                    rows_v.at[slot, pl.ds(off, ln)],
                    gsem[slot],
                )
                for off, ln in splits
            ]

        def store_desc(r, slot):
            return pltpu.make_async_copy(
                rows_v.at[slot], out_hbm.at[base_row + r], osem[slot]
            )

        def posadd(slot):
            @plsc.parallel_loop(0, S, unroll=2)
            def _(j):
                jj = lax.shift_right_logical(j, 3)
                j8 = lax.bitwise_and(j, 7)
                for i in range(D // _LANES):
                    sl = pl.ds(i * _LANES, _LANES)
                    plsc.addupdate(rows_v.at[slot, j, sl], pos_v[jj, j8, sl])

        for d in gather_descs(0, 0):
            d.start()
        for d in gather_descs(1, 1):
            d.start()

        for r in range(rows_per_w):
            slot = r % _NSLOT
            if r >= 1:
                store_desc(r - 1, (r - 1) % _NSLOT).wait()
            if r + 2 < rows_per_w:
                for d in gather_descs(r + 2, (r + 2) % _NSLOT):
                    d.start()
            for d in gather_descs(r, slot):
                d.wait()
            posadd(slot)
            store_desc(r, slot).start()

        store_desc(rows_per_w - 1, (rows_per_w - 1) % _NSLOT).wait()

    return emb(x_flat, token_embedding, pos_t)


# speedup vs baseline: 1.1918x; 1.1918x over previous
import functools

import jax
import jax.numpy as jnp
from jax import lax
from jax.experimental import pallas as pl
from jax.experimental.pallas import tpu as pltpu
import jax.experimental.pallas.tpu_sc as plsc

_NC = 2
_NS = 16
_LANES = 16
_R = 4
_NSLOT = 4
_BLK = 20


def kernel(x, token_embedding, pos_embedding):
    B, S = x.shape
    V, D = token_embedding.shape
    NW = _NC * _NS
    rows_per_w = B // NW
    C = 5
    SC_ = S // C
    n_groups = (rows_per_w // _R) * C
    n_outer = n_groups // _BLK
    ids_per_w = rows_per_w * S

    x_flat = x.astype(jnp.int32).reshape(-1)
    pos_t = pos_embedding.reshape(S // 8, 8, D)

    mesh = plsc.VectorSubcoreMesh(core_axis_name="c", subcore_axis_name="s")

    @functools.partial(
        pl.kernel,
        out_type=jax.ShapeDtypeStruct((B, S, D), jnp.float32),
        mesh=mesh,
        scratch_types=[
            pltpu.VMEM((ids_per_w,), jnp.int32),
            pltpu.VMEM((_NSLOT, _R, SC_, D), jnp.float32),
            pltpu.VMEM((S // 8, 8, D), jnp.float32),
            [pltpu.SemaphoreType.DMA] * _NSLOT,
            [pltpu.SemaphoreType.DMA] * _NSLOT,
            [pltpu.SemaphoreType.DMA] * 2,
        ],
    )
    def emb(x_hbm, tok_hbm, pos_hbm, out_hbm,
            idx_all, rows_v, pos_v, gsem, osem, ssem):
        wid = lax.axis_index("s") * _NC + lax.axis_index("c")
        base_row = wid * rows_per_w
        ids_stage = pltpu.make_async_copy(
            x_hbm.at[pl.ds(wid * ids_per_w, ids_per_w)], idx_all, ssem[0]
        )
        pos_stage = pltpu.make_async_copy(pos_hbm, pos_v, ssem[1])
        ids_stage.start()
        pos_stage.start()
        ids_stage.wait()

        def gather_descs(rg, c, slot):
            return [
                pltpu.make_async_copy(
                    tok_hbm.at[
                        idx_all.at[pl.ds((_R * rg + rr) * S + c * SC_, SC_)]
                    ],
                    rows_v.at[slot, rr],
                    gsem[slot],
                )
                for rr in range(_R)
            ]

        def store_descs(rg, c, slot):
            return [
                pltpu.make_async_copy(
                    rows_v.at[slot, rr],
                    out_hbm.at[base_row + _R * rg + rr, pl.ds(c * SC_, SC_)],
                    osem[slot],
                )
                for rr in range(_R)
            ]

        def posadd(slot, c):
            @plsc.parallel_loop(0, SC_, unroll=2)
            def _(j):
                jj = c * (SC_ // 8) + lax.shift_right_logical(j, 3)
                j8 = lax.bitwise_and(j, 7)
                for i in range(D // _LANES):
                    sl = pl.ds(i * _LANES, _LANES)
                    v = pos_v[jj, j8, sl]
                    for rr in range(_R):
                        plsc.addupdate(rows_v.at[slot, rr, j, sl], v)

        for d in gather_descs(0, 0, 0):
            d.start()
        for d in gather_descs(0, 1, 1):
            d.start()
        pos_stage.wait()

        def outer(t, carry):
            rpo = _BLK // C
            for q in range(_BLK):
                slot = q % _NSLOT
                slot_p = (q + 2) % _NSLOT
                c, c_d, c_p = q % C, (q - 2) % C, (q + 2) % C

                def drain_and_prefetch(t=t, q=q, c_d=c_d, c_p=c_p, slot_p=slot_p):
                    for d in store_descs(rpo * t + (q - 2) // C, c_d, slot_p):
                        d.wait()
                    for d in gather_descs(rpo * t + (q + 2) // C, c_p, slot_p):
                        d.start()

                if q < 2:
                    pl.when(t > 0)(drain_and_prefetch)

                    def prefetch_only(q=q, c_p=c_p, slot_p=slot_p):
                        for d in gather_descs((q + 2) // C, c_p, slot_p):
                            d.start()

                    pl.when(t == 0)(prefetch_only)
                elif q >= _BLK - 2:
                    for d in store_descs(rpo * t + (q - 2) // C, c_d, slot_p):
                        d.wait()

                    def prefetch_next(t=t, q=q, c_p=c_p, slot_p=slot_p):
                        for d in gather_descs(rpo * t + (q + 2) // C, c_p, slot_p):
                            d.start()

                    pl.when(t < n_outer - 1)(prefetch_next)
                else:
                    drain_and_prefetch()

                for d in gather_descs(rpo * t + q // C, c, slot):
                    d.wait()
                posadd(slot, c)
                for d in store_descs(rpo * t + q // C, c, slot):
                    d.start()
            return carry

        lax.fori_loop(0, n_outer, outer, 0)

        for g_last in (n_groups - 2, n_groups - 1):
            for d in store_descs(g_last // C, g_last % C, g_last % _NSLOT):
                d.wait()

    return emb(x_flat, token_embedding, pos_t)
